# Initial kernel scaffold; baseline (speedup 1.0000x reference)
import functools

import jax
import jax.numpy as jnp
from jax import lax
from jax.experimental import pallas as pl
from jax.experimental.pallas import tpu as pltpu
from jax.experimental.pallas import tpu_sc as plsc

N = 10000
E = 320000
D = 128
DE = 16

NC, NS = 2, 16
NT = NC * NS
SUB = 128
CH = 80
E_PAD = NT * CH * SUB
EPT = CH * SUB
N_ACC = N + NS
ROWS_I = N_ACC // NS
ROWS_D = N // NS


def _edge_lin_body(ea_ref, w1_ref, b1_ref, w2_ref, b2_ref, o1_ref, o2_ref):
    ea = ea_ref[...]
    o1_ref[...] = (
        jnp.dot(ea, w1_ref[...], preferred_element_type=jnp.float32) + b1_ref[...]
    )
    o2_ref[...] = (
        jnp.dot(ea, w2_ref[...], preferred_element_type=jnp.float32) + b2_ref[...]
    )


def _edge_lin(ea_pad, w1t, b1, w2t, b2):
    BE = 8192
    grid = (E_PAD // BE,)
    return pl.pallas_call(
        _edge_lin_body,
        grid=grid,
        in_specs=[
            pl.BlockSpec((BE, DE), lambda i: (i, 0)),
            pl.BlockSpec((DE, D), lambda i: (0, 0)),
            pl.BlockSpec((1, D), lambda i: (0, 0)),
            pl.BlockSpec((DE, D), lambda i: (0, 0)),
            pl.BlockSpec((1, D), lambda i: (0, 0)),
        ],
        out_specs=[
            pl.BlockSpec((BE, D), lambda i: (i, 0)),
            pl.BlockSpec((BE, D), lambda i: (i, 0)),
        ],
        out_shape=[
            jax.ShapeDtypeStruct((E_PAD, D), jnp.float32),
            jax.ShapeDtypeStruct((E_PAD, D), jnp.float32),
        ],
        compiler_params=pltpu.CompilerParams(
            dimension_semantics=("arbitrary",),
        ),
    )(ea_pad, w1t, b1, w2t, b2)


def _mlp_body(x_ref, p0_ref, p1_ref, w1_ref, b1_ref, w2_ref, b2_ref, o_ref):
    t = x_ref[...] + p0_ref[...] + p1_ref[...]
    a = jnp.maximum(
        jnp.dot(t, w1_ref[...], preferred_element_type=jnp.float32) + b1_ref[...], 0.0
    )
    h = jnp.dot(a, w2_ref[...], preferred_element_type=jnp.float32) + b2_ref[...]
    o_ref[...] = jnp.maximum(h, 0.0)


def _mlp1(x, p0, p1, w1t, b1, w2t, b2):
    BN = 2000
    grid = (N // BN,)
    return pl.pallas_call(
        _mlp_body,
        grid=grid,
        in_specs=[
            pl.BlockSpec((BN, D), lambda i: (i, 0)),
            pl.BlockSpec((BN, D), lambda i: (i, 0)),
            pl.BlockSpec((BN, D), lambda i: (i, 0)),
            pl.BlockSpec((D, D), lambda i: (0, 0)),
            pl.BlockSpec((1, D), lambda i: (0, 0)),
            pl.BlockSpec((D, D), lambda i: (0, 0)),
            pl.BlockSpec((1, D), lambda i: (0, 0)),
        ],
        out_specs=pl.BlockSpec((BN, D), lambda i: (i, 0)),
        out_shape=jax.ShapeDtypeStruct((N, D), jnp.float32),
        compiler_params=pltpu.CompilerParams(
            dimension_semantics=("arbitrary",),
        ),
    )(x, p0, p1, w1t, b1, w2t, b2)


def _mlp_bn_body(
    h_ref, q0_ref, q1_ref, w1_ref, b1_ref, w2_ref, b2_ref, g_ref, bb_ref, o_ref
):
    t = h_ref[...] + q0_ref[...] + q1_ref[...]
    a = jnp.maximum(
        jnp.dot(t, w1_ref[...], preferred_element_type=jnp.float32) + b1_ref[...], 0.0
    )
    z = jnp.dot(a, w2_ref[...], preferred_element_type=jnp.float32) + b2_ref[...]
    mean = jnp.mean(z, axis=0, keepdims=True)
    var = jnp.mean((z - mean) ** 2, axis=0, keepdims=True)
    o_ref[...] = g_ref[...] * (z - mean) * lax.rsqrt(var + 1e-5) + bb_ref[...]


def _mlp2_bn(h, q0, q1, w1t, b1, w2t, b2, g, bb):
    return pl.pallas_call(
        _mlp_bn_body,
        out_shape=jax.ShapeDtypeStruct((N, D), jnp.float32),
    )(h, q0, q1, w1t, b1, w2t, b2, g, bb)


def _sc_aggr(x, e, src3, dst3, zeros):
    @functools.partial(
        pl.kernel,
        out_type=jax.ShapeDtypeStruct((NC, N, D), jnp.float32),
        mesh=plsc.VectorSubcoreMesh(
            core_axis_name="c", subcore_axis_name="s", num_cores=NC, num_subcores=NS
        ),
        scratch_types=[
            pltpu.VMEM((CH, SUB), jnp.int32),
            pltpu.VMEM((CH, SUB), jnp.int32),
            pltpu.VMEM((SUB, D), jnp.float32),
            pltpu.VMEM((SUB, D), jnp.float32),
            pltpu.VMEM_SHARED((N_ACC, D), jnp.float32),
        ],
    )
    def k(x_hbm, e_hbm, src_hbm, dst_hbm, z_hbm, out_hbm, src_v, dst_v, xs, es, acc):
        c = lax.axis_index("c")
        s = lax.axis_index("s")
        w = c * NS + s
        pltpu.sync_copy(
            z_hbm.at[pl.ds(s * ROWS_I, ROWS_I)], acc.at[pl.ds(s * ROWS_I, ROWS_I)]
        )
        pltpu.sync_copy(src_hbm.at[w], src_v)
        pltpu.sync_copy(dst_hbm.at[w], dst_v)
        plsc.subcore_barrier()
        base = w * EPT

        def chunk(j, carry):
            pltpu.sync_copy(x_hbm.at[src_v.at[j]], xs)
            pltpu.sync_copy(e_hbm.at[pl.ds(base + j * SUB, SUB)], es)

            def row(r, carry2):
                for kq in range(D // 16):
                    sl = pl.ds(kq * 16, 16)
                    es[r, sl] = jnp.maximum(xs[r, sl] + es[r, sl], 0.0)
                return carry2

            lax.fori_loop(0, SUB, row, 0)
            pltpu.sync_copy(es, acc.at[dst_v.at[j]], add=True)
            return carry

        lax.fori_loop(0, CH, chunk, 0)
        plsc.subcore_barrier()
        pltpu.sync_copy(
            acc.at[pl.ds(s * ROWS_D, ROWS_D)], out_hbm.at[c, pl.ds(s * ROWS_D, ROWS_D)]
        )

    return k(x, e, src3, dst3, zeros)


def kernel(x, edge_index, edge_attr, eW1, eb1, m1W1, m1b1, m1W2, m1b2,
           eW2, eb2, m2W1, m2b1, m2W2, m2b2, bn_g, bn_b):
    ep = E_PAD - E
    src3 = jnp.concatenate(
        [edge_index[0], jnp.zeros((ep,), jnp.int32)]
    ).reshape(NT, CH, SUB)
    dst3 = jnp.concatenate(
        [edge_index[1], jnp.full((ep,), N, jnp.int32)]
    ).reshape(NT, CH, SUB)
    ea_pad = jnp.concatenate([edge_attr, jnp.zeros((ep, DE), jnp.float32)])
    zeros = jnp.zeros((N_ACC, D), jnp.float32)

    e1, e2 = _edge_lin(
        ea_pad, eW1.T, eb1.reshape(1, D), eW2.T, eb2.reshape(1, D)
    )
    p = _sc_aggr(x, e1, src3, dst3, zeros)
    h = _mlp1(
        x, p[0], p[1], m1W1.T, m1b1.reshape(1, D), m1W2.T, m1b2.reshape(1, D)
    )
    q = _sc_aggr(h, e2, src3, dst3, zeros)
    out = _mlp2_bn(
        h, q[0], q[1], m2W1.T, m2b1.reshape(1, D), m2W2.T, m2b2.reshape(1, D),
        bn_g.reshape(1, D), bn_b.reshape(1, D),
    )
    return out


# speedup vs baseline: 1.9533x; 1.9533x over previous
import functools

import jax
import jax.numpy as jnp
from jax import lax
from jax.experimental import pallas as pl
from jax.experimental.pallas import tpu as pltpu
from jax.experimental.pallas import tpu_sc as plsc

N = 10000
E = 320000
D = 128
DE = 16

NC, NS = 2, 16
NT = NC * NS
SUB = 128
CH = 80
NG = 2
GC = CH // NG
E_PAD = NT * CH * SUB
EPT = CH * SUB
N_ACC = 10240
ROWS_I = N_ACC // NS


def _edge_lin_body(ea_ref, w1_ref, b1_ref, w2_ref, b2_ref, o1_ref, o2_ref):
    ea = ea_ref[...]
    o1_ref[...] = (
        jnp.dot(ea, w1_ref[...], preferred_element_type=jnp.float32) + b1_ref[...]
    )
    o2_ref[...] = (
        jnp.dot(ea, w2_ref[...], preferred_element_type=jnp.float32) + b2_ref[...]
    )


def _edge_lin(ea_pad, w1t, b1, w2t, b2):
    BE = 8192
    grid = (E_PAD // BE,)
    return pl.pallas_call(
        _edge_lin_body,
        grid=grid,
        in_specs=[
            pl.BlockSpec((BE, DE), lambda i: (i, 0)),
            pl.BlockSpec((DE, D), lambda i: (0, 0)),
            pl.BlockSpec((1, D), lambda i: (0, 0)),
            pl.BlockSpec((DE, D), lambda i: (0, 0)),
            pl.BlockSpec((1, D), lambda i: (0, 0)),
        ],
        out_specs=[
            pl.BlockSpec((BE, D), lambda i: (i, 0)),
            pl.BlockSpec((BE, D), lambda i: (i, 0)),
        ],
        out_shape=[
            jax.ShapeDtypeStruct((E_PAD, D), jnp.float32),
            jax.ShapeDtypeStruct((E_PAD, D), jnp.float32),
        ],
        compiler_params=pltpu.CompilerParams(
            dimension_semantics=("arbitrary",),
        ),
    )(ea_pad, w1t, b1, w2t, b2)


def _mlp_body(x_ref, p0_ref, p1_ref, w1_ref, b1_ref, w2_ref, b2_ref, o_ref):
    t = x_ref[...] + p0_ref[...] + p1_ref[...]
    a = jnp.maximum(
        jnp.dot(t, w1_ref[...], preferred_element_type=jnp.float32) + b1_ref[...], 0.0
    )
    h = jnp.dot(a, w2_ref[...], preferred_element_type=jnp.float32) + b2_ref[...]
    o_ref[...] = jnp.maximum(h, 0.0)


def _mlp1(x, p0, p1, w1t, b1, w2t, b2):
    BN = 2000
    grid = (N // BN,)
    return pl.pallas_call(
        _mlp_body,
        grid=grid,
        in_specs=[
            pl.BlockSpec((BN, D), lambda i: (i, 0)),
            pl.BlockSpec((BN, D), lambda i: (i, 0)),
            pl.BlockSpec((BN, D), lambda i: (i, 0)),
            pl.BlockSpec((D, D), lambda i: (0, 0)),
            pl.BlockSpec((1, D), lambda i: (0, 0)),
            pl.BlockSpec((D, D), lambda i: (0, 0)),
            pl.BlockSpec((1, D), lambda i: (0, 0)),
        ],
        out_specs=pl.BlockSpec((BN, D), lambda i: (i, 0)),
        out_shape=jax.ShapeDtypeStruct((N, D), jnp.float32),
        compiler_params=pltpu.CompilerParams(
            dimension_semantics=("arbitrary",),
        ),
    )(x, p0, p1, w1t, b1, w2t, b2)


def _mlp_bn_body(
    h_ref, q0_ref, q1_ref, w1_ref, b1_ref, w2_ref, b2_ref, g_ref, bb_ref, o_ref
):
    t = h_ref[...] + q0_ref[...] + q1_ref[...]
    a = jnp.maximum(
        jnp.dot(t, w1_ref[...], preferred_element_type=jnp.float32) + b1_ref[...], 0.0
    )
    z = jnp.dot(a, w2_ref[...], preferred_element_type=jnp.float32) + b2_ref[...]
    mean = jnp.mean(z, axis=0, keepdims=True)
    var = jnp.mean((z - mean) ** 2, axis=0, keepdims=True)
    o_ref[...] = g_ref[...] * (z - mean) * lax.rsqrt(var + 1e-5) + bb_ref[...]


def _mlp2_bn(h, q0, q1, w1t, b1, w2t, b2, g, bb):
    return pl.pallas_call(
        _mlp_bn_body,
        grid=(1,),
        in_specs=[
            pl.BlockSpec((N, D), lambda i: (0, 0)),
            pl.BlockSpec((N, D), lambda i: (0, 0)),
            pl.BlockSpec((N, D), lambda i: (0, 0)),
            pl.BlockSpec((D, D), lambda i: (0, 0)),
            pl.BlockSpec((1, D), lambda i: (0, 0)),
            pl.BlockSpec((D, D), lambda i: (0, 0)),
            pl.BlockSpec((1, D), lambda i: (0, 0)),
            pl.BlockSpec((1, D), lambda i: (0, 0)),
            pl.BlockSpec((1, D), lambda i: (0, 0)),
        ],
        out_specs=pl.BlockSpec((N, D), lambda i: (0, 0)),
        out_shape=jax.ShapeDtypeStruct((N, D), jnp.float32),
    )(h, q0, q1, w1t, b1, w2t, b2, g, bb)


def _sc_aggr(x, e, src3, dst3, zeros):
    @functools.partial(
        pl.kernel,
        out_type=jax.ShapeDtypeStruct((NC, N_ACC, D), jnp.float32),
        mesh=plsc.VectorSubcoreMesh(
            core_axis_name="c", subcore_axis_name="s", num_cores=NC, num_subcores=NS
        ),
        scratch_types=[
            pltpu.VMEM((GC, SUB), jnp.int32),
            pltpu.VMEM((GC, SUB), jnp.int32),
            pltpu.VMEM((SUB, D), jnp.float32),
            pltpu.VMEM((SUB, D), jnp.float32),
            pltpu.VMEM_SHARED((N_ACC, D), jnp.float32),
        ],
    )
    def k(x_hbm, e_hbm, src_hbm, dst_hbm, z_hbm, out_hbm, src_v, dst_v, xs, es, acc):
        c = lax.axis_index("c")
        s = lax.axis_index("s")
        w = c * NS + s
        pltpu.sync_copy(
            z_hbm.at[pl.ds(s * ROWS_I, ROWS_I)], acc.at[pl.ds(s * ROWS_I, ROWS_I)]
        )
        plsc.subcore_barrier()
        base = w * EPT

        def group(g, carry0):
            pltpu.sync_copy(src_hbm.at[w, pl.ds(g * GC, GC)], src_v)
            pltpu.sync_copy(dst_hbm.at[w, pl.ds(g * GC, GC)], dst_v)

            def chunk(j, carry):
                pltpu.sync_copy(x_hbm.at[src_v.at[j]], xs)
                pltpu.sync_copy(
                    e_hbm.at[pl.ds(base + (g * GC + j) * SUB, SUB)], es
                )

                def row(r, carry2):
                    for kq in range(D // 16):
                        sl = pl.ds(kq * 16, 16)
                        es[r, sl] = jnp.maximum(xs[r, sl] + es[r, sl], 0.0)
                    return carry2

                lax.fori_loop(0, SUB, row, 0)
                pltpu.sync_copy(es, acc.at[dst_v.at[j]], add=True)
                return carry

            lax.fori_loop(0, GC, chunk, 0)
            return carry0

        lax.fori_loop(0, NG, group, 0)
        plsc.subcore_barrier()
        pltpu.sync_copy(
            acc.at[pl.ds(s * ROWS_I, ROWS_I)], out_hbm.at[c, pl.ds(s * ROWS_I, ROWS_I)]
        )

    return k(x, e, src3, dst3, zeros)


def kernel(x, edge_index, edge_attr, eW1, eb1, m1W1, m1b1, m1W2, m1b2,
           eW2, eb2, m2W1, m2b1, m2W2, m2b2, bn_g, bn_b):
    ep = E_PAD - E
    src3 = jnp.concatenate(
        [edge_index[0], jnp.zeros((ep,), jnp.int32)]
    ).reshape(NT, CH, SUB)
    dst3 = jnp.concatenate(
        [edge_index[1], jnp.full((ep,), N, jnp.int32)]
    ).reshape(NT, CH, SUB)
    ea_pad = jnp.concatenate([edge_attr, jnp.zeros((ep, DE), jnp.float32)])
    zeros = jnp.zeros((N_ACC, D), jnp.float32)

    e1, e2 = _edge_lin(
        ea_pad, eW1.T, eb1.reshape(1, D), eW2.T, eb2.reshape(1, D)
    )
    p = _sc_aggr(x, e1, src3, dst3, zeros)
    h = _mlp1(
        x, p[0], p[1], m1W1.T, m1b1.reshape(1, D), m1W2.T, m1b2.reshape(1, D)
    )
    q = _sc_aggr(h, e2, src3, dst3, zeros)
    out = _mlp2_bn(
        h, q[0], q[1], m2W1.T, m2b1.reshape(1, D), m2W2.T, m2b2.reshape(1, D),
        bn_g.reshape(1, D), bn_b.reshape(1, D),
    )
    return out


# speedup vs baseline: 1.9561x; 1.0014x over previous
import functools

import jax
import jax.numpy as jnp
from jax import lax
from jax.experimental import pallas as pl
from jax.experimental.pallas import tpu as pltpu
from jax.experimental.pallas import tpu_sc as plsc

N = 10000
E = 320000
D = 128
DE = 16

NC, NS = 2, 16
NT = NC * NS
SUB = 128
CH = 80
NG = 2
GC = CH // NG
E_PAD = NT * CH * SUB
EPT = CH * SUB
N_ACC = 10240
ROWS_I = N_ACC // NS


def _edge_lin_body(ea_ref, w1_ref, b1_ref, w2_ref, b2_ref, o1_ref, o2_ref):
    ea = ea_ref[...]
    o1_ref[...] = (
        jnp.dot(ea, w1_ref[...], preferred_element_type=jnp.float32) + b1_ref[...]
    )
    o2_ref[...] = (
        jnp.dot(ea, w2_ref[...], preferred_element_type=jnp.float32) + b2_ref[...]
    )


def _edge_lin(ea_pad, w1t, b1, w2t, b2):
    BE = 8192
    grid = (E_PAD // BE,)
    return pl.pallas_call(
        _edge_lin_body,
        grid=grid,
        in_specs=[
            pl.BlockSpec((BE, DE), lambda i: (i, 0)),
            pl.BlockSpec((DE, D), lambda i: (0, 0)),
            pl.BlockSpec((1, D), lambda i: (0, 0)),
            pl.BlockSpec((DE, D), lambda i: (0, 0)),
            pl.BlockSpec((1, D), lambda i: (0, 0)),
        ],
        out_specs=[
            pl.BlockSpec((BE, D), lambda i: (i, 0)),
            pl.BlockSpec((BE, D), lambda i: (i, 0)),
        ],
        out_shape=[
            jax.ShapeDtypeStruct((E_PAD, D), jnp.float32),
            jax.ShapeDtypeStruct((E_PAD, D), jnp.float32),
        ],
        compiler_params=pltpu.CompilerParams(
            dimension_semantics=("arbitrary",),
        ),
    )(ea_pad, w1t, b1, w2t, b2)


def _mlp_body(x_ref, p0_ref, p1_ref, w1_ref, b1_ref, w2_ref, b2_ref, o_ref):
    t = x_ref[...] + p0_ref[...] + p1_ref[...]
    a = jnp.maximum(
        jnp.dot(t, w1_ref[...], preferred_element_type=jnp.float32) + b1_ref[...], 0.0
    )
    h = jnp.dot(a, w2_ref[...], preferred_element_type=jnp.float32) + b2_ref[...]
    o_ref[...] = jnp.maximum(h, 0.0)


def _mlp1(x, p0, p1, w1t, b1, w2t, b2):
    BN = 2000
    grid = (N // BN,)
    return pl.pallas_call(
        _mlp_body,
        grid=grid,
        in_specs=[
            pl.BlockSpec((BN, D), lambda i: (i, 0)),
            pl.BlockSpec((BN, D), lambda i: (i, 0)),
            pl.BlockSpec((BN, D), lambda i: (i, 0)),
            pl.BlockSpec((D, D), lambda i: (0, 0)),
            pl.BlockSpec((1, D), lambda i: (0, 0)),
            pl.BlockSpec((D, D), lambda i: (0, 0)),
            pl.BlockSpec((1, D), lambda i: (0, 0)),
        ],
        out_specs=pl.BlockSpec((BN, D), lambda i: (i, 0)),
        out_shape=jax.ShapeDtypeStruct((N, D), jnp.float32),
        compiler_params=pltpu.CompilerParams(
            dimension_semantics=("arbitrary",),
        ),
    )(x, p0, p1, w1t, b1, w2t, b2)


def _mlp_bn_body(
    h_ref, q0_ref, q1_ref, w1_ref, b1_ref, w2_ref, b2_ref, g_ref, bb_ref, o_ref
):
    t = h_ref[...] + q0_ref[...] + q1_ref[...]
    a = jnp.maximum(
        jnp.dot(t, w1_ref[...], preferred_element_type=jnp.float32) + b1_ref[...], 0.0
    )
    z = jnp.dot(a, w2_ref[...], preferred_element_type=jnp.float32) + b2_ref[...]
    mean = jnp.mean(z, axis=0, keepdims=True)
    var = jnp.mean((z - mean) ** 2, axis=0, keepdims=True)
    o_ref[...] = g_ref[...] * (z - mean) * lax.rsqrt(var + 1e-5) + bb_ref[...]


def _mlp2_bn(h, q0, q1, w1t, b1, w2t, b2, g, bb):
    return pl.pallas_call(
        _mlp_bn_body,
        grid=(1,),
        in_specs=[
            pl.BlockSpec((N, D), lambda i: (0, 0)),
            pl.BlockSpec((N, D), lambda i: (0, 0)),
            pl.BlockSpec((N, D), lambda i: (0, 0)),
            pl.BlockSpec((D, D), lambda i: (0, 0)),
            pl.BlockSpec((1, D), lambda i: (0, 0)),
            pl.BlockSpec((D, D), lambda i: (0, 0)),
            pl.BlockSpec((1, D), lambda i: (0, 0)),
            pl.BlockSpec((1, D), lambda i: (0, 0)),
            pl.BlockSpec((1, D), lambda i: (0, 0)),
        ],
        out_specs=pl.BlockSpec((N, D), lambda i: (0, 0)),
        out_shape=jax.ShapeDtypeStruct((N, D), jnp.float32),
    )(h, q0, q1, w1t, b1, w2t, b2, g, bb)


def _sc_aggr(x, e, src3, dst3, zeros):
    @functools.partial(
        pl.kernel,
        out_type=jax.ShapeDtypeStruct((NC, N_ACC, D), jnp.float32),
        mesh=plsc.VectorSubcoreMesh(
            core_axis_name="c", subcore_axis_name="s", num_cores=NC, num_subcores=NS
        ),
        scratch_types=[
            pltpu.VMEM((GC, SUB), jnp.int32),
            pltpu.VMEM((GC, SUB), jnp.int32),
            pltpu.VMEM((SUB, D), jnp.float32),
            pltpu.VMEM((SUB, D), jnp.float32),
            pltpu.VMEM_SHARED((N_ACC, D), jnp.float32),
        ],
    )
    def k(x_hbm, e_hbm, src_hbm, dst_hbm, z_hbm, out_hbm, src_v, dst_v, xs, es, acc):
        c = lax.axis_index("c")
        s = lax.axis_index("s")
        w = c * NS + s
        pltpu.sync_copy(
            z_hbm.at[pl.ds(s * ROWS_I, ROWS_I)], acc.at[pl.ds(s * ROWS_I, ROWS_I)]
        )
        plsc.subcore_barrier()
        base = w * EPT

        def group(g, carry0):
            pltpu.sync_copy(src_hbm.at[w, pl.ds(g * GC, GC)], src_v)
            pltpu.sync_copy(dst_hbm.at[w, pl.ds(g * GC, GC)], dst_v)

            def chunk(j, carry):
                pltpu.sync_copy(x_hbm.at[src_v.at[j]], xs)
                pltpu.sync_copy(
                    e_hbm.at[pl.ds(base + (g * GC + j) * SUB, SUB)], es
                )

                def row(r, carry2):
                    for kq in range(D // 16):
                        sl = pl.ds(kq * 16, 16)
                        es[r, sl] = jnp.maximum(xs[r, sl] + es[r, sl], 0.0)
                    return carry2

                lax.fori_loop(0, SUB, row, 0)
                pltpu.sync_copy(es, acc.at[dst_v.at[j]], add=True)
                return carry

            lax.fori_loop(0, GC, chunk, 0)
            return carry0

        lax.fori_loop(0, NG, group, 0)
        plsc.subcore_barrier()
        pltpu.sync_copy(
            acc.at[pl.ds(s * ROWS_I, ROWS_I)], out_hbm.at[c, pl.ds(s * ROWS_I, ROWS_I)]
        )

    return k(x, e, src3, dst3, zeros)


def kernel(x, edge_index, edge_attr, eW1, eb1, m1W1, m1b1, m1W2, m1b2,
           eW2, eb2, m2W1, m2b1, m2W2, m2b2, bn_g, bn_b):
    ep = E_PAD - E
    src3 = jnp.concatenate(
        [edge_index[0], jnp.zeros((ep,), jnp.int32)]
    ).reshape(NT, CH, SUB)
    pad_dst = N + (jnp.arange(ep, dtype=jnp.int32) % (N_ACC - N))
    dst3 = jnp.concatenate([edge_index[1], pad_dst]).reshape(NT, CH, SUB)
    ea_pad = jnp.concatenate([edge_attr, jnp.zeros((ep, DE), jnp.float32)])
    zeros = jnp.zeros((N_ACC, D), jnp.float32)

    e1, e2 = _edge_lin(
        ea_pad, eW1.T, eb1.reshape(1, D), eW2.T, eb2.reshape(1, D)
    )
    p = _sc_aggr(x, e1, src3, dst3, zeros)
    h = _mlp1(
        x, p[0], p[1], m1W1.T, m1b1.reshape(1, D), m1W2.T, m1b2.reshape(1, D)
    )
    q = _sc_aggr(h, e2, src3, dst3, zeros)
    out = _mlp2_bn(
        h, q[0], q[1], m2W1.T, m2b1.reshape(1, D), m2W2.T, m2b2.reshape(1, D),
        bn_g.reshape(1, D), bn_b.reshape(1, D),
    )
    return out


# speedup vs baseline: 2.4837x; 1.2697x over previous
import functools

import jax
import jax.numpy as jnp
from jax import lax
from jax.experimental import pallas as pl
from jax.experimental.pallas import tpu as pltpu
from jax.experimental.pallas import tpu_sc as plsc

N = 10000
E = 320000
D = 128
DE = 16

NC, NS = 2, 16
NT = NC * NS
SUB = 64
CH = 160
NG = 4
GC = CH // NG
E_PAD = NT * CH * SUB
EPT = CH * SUB
N_ACC = 10240
ROWS_I = N_ACC // NS


def _edge_lin_body(ea_ref, w1_ref, b1_ref, w2_ref, b2_ref, o1_ref, o2_ref):
    ea = ea_ref[...]
    o1_ref[...] = (
        jnp.dot(ea, w1_ref[...], preferred_element_type=jnp.float32) + b1_ref[...]
    )
    o2_ref[...] = (
        jnp.dot(ea, w2_ref[...], preferred_element_type=jnp.float32) + b2_ref[...]
    )


def _edge_lin(ea_pad, w1t, b1, w2t, b2):
    BE = 8192
    grid = (E_PAD // BE,)
    return pl.pallas_call(
        _edge_lin_body,
        grid=grid,
        in_specs=[
            pl.BlockSpec((BE, DE), lambda i: (i, 0)),
            pl.BlockSpec((DE, D), lambda i: (0, 0)),
            pl.BlockSpec((1, D), lambda i: (0, 0)),
            pl.BlockSpec((DE, D), lambda i: (0, 0)),
            pl.BlockSpec((1, D), lambda i: (0, 0)),
        ],
        out_specs=[
            pl.BlockSpec((BE, D), lambda i: (i, 0)),
            pl.BlockSpec((BE, D), lambda i: (i, 0)),
        ],
        out_shape=[
            jax.ShapeDtypeStruct((E_PAD, D), jnp.float32),
            jax.ShapeDtypeStruct((E_PAD, D), jnp.float32),
        ],
        compiler_params=pltpu.CompilerParams(
            dimension_semantics=("arbitrary",),
        ),
    )(ea_pad, w1t, b1, w2t, b2)


def _mlp_body(x_ref, p0_ref, p1_ref, w1_ref, b1_ref, w2_ref, b2_ref, o_ref):
    t = x_ref[...] + p0_ref[...] + p1_ref[...]
    a = jnp.maximum(
        jnp.dot(t, w1_ref[...], preferred_element_type=jnp.float32) + b1_ref[...], 0.0
    )
    h = jnp.dot(a, w2_ref[...], preferred_element_type=jnp.float32) + b2_ref[...]
    o_ref[...] = jnp.maximum(h, 0.0)


def _mlp1(x, p0, p1, w1t, b1, w2t, b2):
    BN = 2000
    grid = (N // BN,)
    return pl.pallas_call(
        _mlp_body,
        grid=grid,
        in_specs=[
            pl.BlockSpec((BN, D), lambda i: (i, 0)),
            pl.BlockSpec((BN, D), lambda i: (i, 0)),
            pl.BlockSpec((BN, D), lambda i: (i, 0)),
            pl.BlockSpec((D, D), lambda i: (0, 0)),
            pl.BlockSpec((1, D), lambda i: (0, 0)),
            pl.BlockSpec((D, D), lambda i: (0, 0)),
            pl.BlockSpec((1, D), lambda i: (0, 0)),
        ],
        out_specs=pl.BlockSpec((BN, D), lambda i: (i, 0)),
        out_shape=jax.ShapeDtypeStruct((N, D), jnp.float32),
        compiler_params=pltpu.CompilerParams(
            dimension_semantics=("arbitrary",),
        ),
    )(x, p0, p1, w1t, b1, w2t, b2)


def _mlp_bn_body(
    h_ref, q0_ref, q1_ref, w1_ref, b1_ref, w2_ref, b2_ref, g_ref, bb_ref, o_ref
):
    t = h_ref[...] + q0_ref[...] + q1_ref[...]
    a = jnp.maximum(
        jnp.dot(t, w1_ref[...], preferred_element_type=jnp.float32) + b1_ref[...], 0.0
    )
    z = jnp.dot(a, w2_ref[...], preferred_element_type=jnp.float32) + b2_ref[...]
    mean = jnp.mean(z, axis=0, keepdims=True)
    var = jnp.mean((z - mean) ** 2, axis=0, keepdims=True)
    o_ref[...] = g_ref[...] * (z - mean) * lax.rsqrt(var + 1e-5) + bb_ref[...]


def _mlp2_bn(h, q0, q1, w1t, b1, w2t, b2, g, bb):
    return pl.pallas_call(
        _mlp_bn_body,
        grid=(1,),
        in_specs=[
            pl.BlockSpec((N, D), lambda i: (0, 0)),
            pl.BlockSpec((N, D), lambda i: (0, 0)),
            pl.BlockSpec((N, D), lambda i: (0, 0)),
            pl.BlockSpec((D, D), lambda i: (0, 0)),
            pl.BlockSpec((1, D), lambda i: (0, 0)),
            pl.BlockSpec((D, D), lambda i: (0, 0)),
            pl.BlockSpec((1, D), lambda i: (0, 0)),
            pl.BlockSpec((1, D), lambda i: (0, 0)),
            pl.BlockSpec((1, D), lambda i: (0, 0)),
        ],
        out_specs=pl.BlockSpec((N, D), lambda i: (0, 0)),
        out_shape=jax.ShapeDtypeStruct((N, D), jnp.float32),
    )(h, q0, q1, w1t, b1, w2t, b2, g, bb)


def _sc_aggr(x, e, src3, dst3, zeros):
    @functools.partial(
        pl.kernel,
        out_type=jax.ShapeDtypeStruct((NC, N_ACC, D), jnp.float32),
        mesh=plsc.VectorSubcoreMesh(
            core_axis_name="c", subcore_axis_name="s", num_cores=NC, num_subcores=NS
        ),
        scratch_types=[
            pltpu.VMEM((GC, SUB), jnp.int32),
            pltpu.VMEM((GC, SUB), jnp.int32),
            pltpu.VMEM((SUB, D), jnp.float32),
            pltpu.VMEM((SUB, D), jnp.float32),
            pltpu.VMEM((SUB, D), jnp.float32),
            pltpu.VMEM((SUB, D), jnp.float32),
            pltpu.SemaphoreType.DMA,
            pltpu.SemaphoreType.DMA,
            pltpu.VMEM_SHARED((N_ACC, D), jnp.float32),
        ],
    )
    def k(x_hbm, e_hbm, src_hbm, dst_hbm, z_hbm, out_hbm,
          src_v, dst_v, xs0, es0, xs1, es1, sem0, sem1, acc):
        c = lax.axis_index("c")
        s = lax.axis_index("s")
        w = c * NS + s
        pltpu.sync_copy(
            z_hbm.at[pl.ds(s * ROWS_I, ROWS_I)], acc.at[pl.ds(s * ROWS_I, ROWS_I)]
        )
        plsc.subcore_barrier()
        base = w * EPT

        def body(g):
            ebase = base + g * GC * SUB

            def start(j, xsb, esb, sem):
                pltpu.async_copy(x_hbm.at[src_v.at[j]], xsb, sem)
                pltpu.async_copy(e_hbm.at[pl.ds(ebase + j * SUB, SUB)], esb, sem)

            def finish(j, xsb, esb, sem):
                pltpu.make_async_copy(x_hbm.at[src_v.at[j]], xsb, sem).wait()
                pltpu.make_async_copy(
                    e_hbm.at[pl.ds(ebase + j * SUB, SUB)], esb, sem
                ).wait()

            def work(j, xsb, esb):
                def row(r, carry2):
                    for kq in range(D // 16):
                        sl = pl.ds(kq * 16, 16)
                        esb[r, sl] = jnp.maximum(xsb[r, sl] + esb[r, sl], 0.0)
                    return carry2

                lax.fori_loop(0, SUB, row, 0)
                pltpu.sync_copy(esb, acc.at[dst_v.at[j]], add=True)

            pltpu.sync_copy(src_hbm.at[w, pl.ds(g * GC, GC)], src_v)
            pltpu.sync_copy(dst_hbm.at[w, pl.ds(g * GC, GC)], dst_v)
            start(0, xs0, es0, sem0)

            def pair(j2, carry):
                j = 2 * j2
                start(j + 1, xs1, es1, sem1)
                finish(j, xs0, es0, sem0)
                work(j, xs0, es0)
                start(j + 2, xs0, es0, sem0)
                finish(j + 1, xs1, es1, sem1)
                work(j + 1, xs1, es1)
                return carry

            lax.fori_loop(0, GC // 2 - 1, pair, 0)
            start(GC - 1, xs1, es1, sem1)
            finish(GC - 2, xs0, es0, sem0)
            work(GC - 2, xs0, es0)
            finish(GC - 1, xs1, es1, sem1)
            work(GC - 1, xs1, es1)

        for g in range(NG):
            body(g)
        plsc.subcore_barrier()
        pltpu.sync_copy(
            acc.at[pl.ds(s * ROWS_I, ROWS_I)], out_hbm.at[c, pl.ds(s * ROWS_I, ROWS_I)]
        )

    return k(x, e, src3, dst3, zeros)


def kernel(x, edge_index, edge_attr, eW1, eb1, m1W1, m1b1, m1W2, m1b2,
           eW2, eb2, m2W1, m2b1, m2W2, m2b2, bn_g, bn_b):
    ep = E_PAD - E
    src3 = jnp.concatenate(
        [edge_index[0], jnp.zeros((ep,), jnp.int32)]
    ).reshape(NT, CH, SUB)
    pad_dst = N + (jnp.arange(ep, dtype=jnp.int32) % (N_ACC - N))
    dst3 = jnp.concatenate([edge_index[1], pad_dst]).reshape(NT, CH, SUB)
    ea_pad = jnp.concatenate([edge_attr, jnp.zeros((ep, DE), jnp.float32)])
    zeros = jnp.zeros((N_ACC, D), jnp.float32)

    e1, e2 = _edge_lin(
        ea_pad, eW1.T, eb1.reshape(1, D), eW2.T, eb2.reshape(1, D)
    )
    p = _sc_aggr(x, e1, src3, dst3, zeros)
    h = _mlp1(
        x, p[0], p[1], m1W1.T, m1b1.reshape(1, D), m1W2.T, m1b2.reshape(1, D)
    )
    q = _sc_aggr(h, e2, src3, dst3, zeros)
    out = _mlp2_bn(
        h, q[0], q[1], m2W1.T, m2b1.reshape(1, D), m2W2.T, m2b2.reshape(1, D),
        bn_g.reshape(1, D), bn_b.reshape(1, D),
    )
    return out


# speedup vs baseline: 2.6600x; 1.0710x over previous
import functools

import jax
import jax.numpy as jnp
from jax import lax
from jax.experimental import pallas as pl
from jax.experimental.pallas import tpu as pltpu
from jax.experimental.pallas import tpu_sc as plsc

N = 10000
E = 320000
D = 128
DE = 16

NC, NS = 2, 16
NT = NC * NS
SUB = 64
GC = 32
CH0 = 224
CH1 = 96
TOT_CH = NS * (CH0 + CH1)
E_PAD = TOT_CH * SUB
N_ACC = 10240
ROWS_I = N_ACC // NS


def _edge_lin_body(ea_ref, w1_ref, b1_ref, w2_ref, b2_ref, o1_ref, o2_ref):
    ea = ea_ref[...]
    o1_ref[...] = (
        jnp.dot(ea, w1_ref[...], preferred_element_type=jnp.float32) + b1_ref[...]
    )
    o2_ref[...] = (
        jnp.dot(ea, w2_ref[...], preferred_element_type=jnp.float32) + b2_ref[...]
    )


def _edge_lin(ea_pad, w1t, b1, w2t, b2):
    BE = 8192
    grid = (E_PAD // BE,)
    return pl.pallas_call(
        _edge_lin_body,
        grid=grid,
        in_specs=[
            pl.BlockSpec((BE, DE), lambda i: (i, 0)),
            pl.BlockSpec((DE, D), lambda i: (0, 0)),
            pl.BlockSpec((1, D), lambda i: (0, 0)),
            pl.BlockSpec((DE, D), lambda i: (0, 0)),
            pl.BlockSpec((1, D), lambda i: (0, 0)),
        ],
        out_specs=[
            pl.BlockSpec((BE, D), lambda i: (i, 0)),
            pl.BlockSpec((BE, D), lambda i: (i, 0)),
        ],
        out_shape=[
            jax.ShapeDtypeStruct((E_PAD, D), jnp.float32),
            jax.ShapeDtypeStruct((E_PAD, D), jnp.float32),
        ],
        compiler_params=pltpu.CompilerParams(
            dimension_semantics=("arbitrary",),
        ),
    )(ea_pad, w1t, b1, w2t, b2)


def _mlp_body(x_ref, p0_ref, p1_ref, w1_ref, b1_ref, w2_ref, b2_ref, o_ref):
    t = x_ref[...] + p0_ref[...] + p1_ref[...]
    a = jnp.maximum(
        jnp.dot(t, w1_ref[...], preferred_element_type=jnp.float32) + b1_ref[...], 0.0
    )
    h = jnp.dot(a, w2_ref[...], preferred_element_type=jnp.float32) + b2_ref[...]
    o_ref[...] = jnp.maximum(h, 0.0)


def _mlp1(x, p0, p1, w1t, b1, w2t, b2):
    BN = 2000
    grid = (N // BN,)
    return pl.pallas_call(
        _mlp_body,
        grid=grid,
        in_specs=[
            pl.BlockSpec((BN, D), lambda i: (i, 0)),
            pl.BlockSpec((BN, D), lambda i: (i, 0)),
            pl.BlockSpec((BN, D), lambda i: (i, 0)),
            pl.BlockSpec((D, D), lambda i: (0, 0)),
            pl.BlockSpec((1, D), lambda i: (0, 0)),
            pl.BlockSpec((D, D), lambda i: (0, 0)),
            pl.BlockSpec((1, D), lambda i: (0, 0)),
        ],
        out_specs=pl.BlockSpec((BN, D), lambda i: (i, 0)),
        out_shape=jax.ShapeDtypeStruct((N, D), jnp.float32),
        compiler_params=pltpu.CompilerParams(
            dimension_semantics=("arbitrary",),
        ),
    )(x, p0, p1, w1t, b1, w2t, b2)


def _mlp_bn_body(
    h_ref, q0_ref, q1_ref, w1_ref, b1_ref, w2_ref, b2_ref, g_ref, bb_ref, o_ref
):
    t = h_ref[...] + q0_ref[...] + q1_ref[...]
    a = jnp.maximum(
        jnp.dot(t, w1_ref[...], preferred_element_type=jnp.float32) + b1_ref[...], 0.0
    )
    z = jnp.dot(a, w2_ref[...], preferred_element_type=jnp.float32) + b2_ref[...]
    mean = jnp.mean(z, axis=0, keepdims=True)
    var = jnp.mean((z - mean) ** 2, axis=0, keepdims=True)
    o_ref[...] = g_ref[...] * (z - mean) * lax.rsqrt(var + 1e-5) + bb_ref[...]


def _mlp2_bn(h, q0, q1, w1t, b1, w2t, b2, g, bb):
    return pl.pallas_call(
        _mlp_bn_body,
        grid=(1,),
        in_specs=[
            pl.BlockSpec((N, D), lambda i: (0, 0)),
            pl.BlockSpec((N, D), lambda i: (0, 0)),
            pl.BlockSpec((N, D), lambda i: (0, 0)),
            pl.BlockSpec((D, D), lambda i: (0, 0)),
            pl.BlockSpec((1, D), lambda i: (0, 0)),
            pl.BlockSpec((D, D), lambda i: (0, 0)),
            pl.BlockSpec((1, D), lambda i: (0, 0)),
            pl.BlockSpec((1, D), lambda i: (0, 0)),
            pl.BlockSpec((1, D), lambda i: (0, 0)),
        ],
        out_specs=pl.BlockSpec((N, D), lambda i: (0, 0)),
        out_shape=jax.ShapeDtypeStruct((N, D), jnp.float32),
    )(h, q0, q1, w1t, b1, w2t, b2, g, bb)


def _sc_aggr(x, e, src3, dst3, zeros):
    @functools.partial(
        pl.kernel,
        out_type=jax.ShapeDtypeStruct((NC, N_ACC, D), jnp.float32),
        mesh=plsc.VectorSubcoreMesh(
            core_axis_name="c", subcore_axis_name="s", num_cores=NC, num_subcores=NS
        ),
        scratch_types=[
            pltpu.VMEM((GC, SUB), jnp.int32),
            pltpu.VMEM((GC, SUB), jnp.int32),
            pltpu.VMEM((SUB, D), jnp.float32),
            pltpu.VMEM((SUB, D), jnp.float32),
            pltpu.VMEM((SUB, D), jnp.float32),
            pltpu.VMEM((SUB, D), jnp.float32),
            pltpu.SemaphoreType.DMA,
            pltpu.SemaphoreType.DMA,
            pltpu.VMEM_SHARED((N_ACC, D), jnp.float32),
        ],
    )
    def k(x_hbm, e_hbm, src_hbm, dst_hbm, z_hbm, out_hbm,
          src_v, dst_v, xs0, es0, xs1, es1, sem0, sem1, acc):
        c = lax.axis_index("c")
        s = lax.axis_index("s")
        pltpu.sync_copy(
            z_hbm.at[pl.ds(s * ROWS_I, ROWS_I)], acc.at[pl.ds(s * ROWS_I, ROWS_I)]
        )
        plsc.subcore_barrier()
        chc = jnp.where(c == 0, CH0, CH1)
        cbase = c * NS * CH0 + s * chc

        def body(g, carry0):
            gbase = cbase + g * GC
            ebase = gbase * SUB

            def start(j, xsb, esb, sem):
                pltpu.async_copy(x_hbm.at[src_v.at[j]], xsb, sem)
                pltpu.async_copy(e_hbm.at[pl.ds(ebase + j * SUB, SUB)], esb, sem)

            def finish(j, xsb, esb, sem):
                pltpu.make_async_copy(x_hbm.at[src_v.at[j]], xsb, sem).wait()
                pltpu.make_async_copy(
                    e_hbm.at[pl.ds(ebase + j * SUB, SUB)], esb, sem
                ).wait()

            def work(j, xsb, esb):
                def row(r, carry2):
                    for kq in range(D // 16):
                        sl = pl.ds(kq * 16, 16)
                        esb[r, sl] = jnp.maximum(xsb[r, sl] + esb[r, sl], 0.0)
                    return carry2

                lax.fori_loop(0, SUB, row, 0)
                pltpu.sync_copy(esb, acc.at[dst_v.at[j]], add=True)

            pltpu.sync_copy(src_hbm.at[pl.ds(gbase, GC)], src_v)
            pltpu.sync_copy(dst_hbm.at[pl.ds(gbase, GC)], dst_v)
            start(0, xs0, es0, sem0)

            def pair(j2, carry):
                j = 2 * j2
                start(j + 1, xs1, es1, sem1)
                finish(j, xs0, es0, sem0)
                work(j, xs0, es0)
                start(j + 2, xs0, es0, sem0)
                finish(j + 1, xs1, es1, sem1)
                work(j + 1, xs1, es1)
                return carry

            lax.fori_loop(0, GC // 2 - 1, pair, 0)
            start(GC - 1, xs1, es1, sem1)
            finish(GC - 2, xs0, es0, sem0)
            work(GC - 2, xs0, es0)
            finish(GC - 1, xs1, es1, sem1)
            work(GC - 1, xs1, es1)
            return carry0

        lax.fori_loop(0, chc // GC, body, 0)
        plsc.subcore_barrier()
        pltpu.sync_copy(
            acc.at[pl.ds(s * ROWS_I, ROWS_I)], out_hbm.at[c, pl.ds(s * ROWS_I, ROWS_I)]
        )

    return k(x, e, src3, dst3, zeros)


def kernel(x, edge_index, edge_attr, eW1, eb1, m1W1, m1b1, m1W2, m1b2,
           eW2, eb2, m2W1, m2b1, m2W2, m2b2, bn_g, bn_b):
    ep = E_PAD - E
    src3 = jnp.concatenate(
        [edge_index[0], jnp.zeros((ep,), jnp.int32)]
    ).reshape(TOT_CH, SUB)
    pad_dst = N + (jnp.arange(ep, dtype=jnp.int32) % (N_ACC - N))
    dst3 = jnp.concatenate([edge_index[1], pad_dst]).reshape(TOT_CH, SUB)
    ea_pad = jnp.concatenate([edge_attr, jnp.zeros((ep, DE), jnp.float32)])
    zeros = jnp.zeros((N_ACC, D), jnp.float32)

    e1, e2 = _edge_lin(
        ea_pad, eW1.T, eb1.reshape(1, D), eW2.T, eb2.reshape(1, D)
    )
    p = _sc_aggr(x, e1, src3, dst3, zeros)
    h = _mlp1(
        x, p[0], p[1], m1W1.T, m1b1.reshape(1, D), m1W2.T, m1b2.reshape(1, D)
    )
    q = _sc_aggr(h, e2, src3, dst3, zeros)
    out = _mlp2_bn(
        h, q[0], q[1], m2W1.T, m2b1.reshape(1, D), m2W2.T, m2b2.reshape(1, D),
        bn_g.reshape(1, D), bn_b.reshape(1, D),
    )
    return out
